# baseline (speedup 1.0000x reference)
import functools

import jax
import jax.numpy as jnp
from jax.experimental import pallas as pl
from jax.experimental.pallas import tpu as pltpu

_SLOPE = 0.01
_TEMP = 0.5
_EPS = 1e-7


def _lrelu(y):
    return jnp.maximum(y, y * jnp.asarray(_SLOPE, y.dtype))


def _conv0_chunks():
    out = []
    for l0 in range(0, 127, 16):
        nl = min(16, 127 - l0)
        out.append((2 * l0, min(2 * (l0 + nl - 1) + 3, 256),
                    16 * l0, 16 * nl))
    return out


def _conv1_chunks():
    out = []
    for l0 in range(0, 63, 4):
        nl = min(4, 63 - l0)
        out.append((32 * l0, min(32 * (l0 + nl) + 16, 2032),
                    32 * l0, 32 * nl))
    return out


def _convt0_chunks():
    out = []
    for o0 in range(0, 127, 8):
        no = min(8, 127 - o0)
        lmin = max(0, -((2 - o0) // 2))
        lmax = min(62, (o0 + no - 1) // 2)
        out.append((32 * lmin, 32 * (lmax + 1), 16 * o0, 16 * no))
    return out


_CONV0_CHUNKS = _conv0_chunks()
_CONV1_CHUNKS = _conv1_chunks()
_CONVT0_CHUNKS = _convt0_chunks()
_CONVT1_CHUNKS = [(0, 1024, 0, 128), (1008, 2032, 128, 128)]


def _fused_body(x_ref, u_ref,
                wc0_ref, bc0_ref, wc1_ref, bc1_ref,
                wf1_ref, bf1_ref, wf2_ref, bf2_ref,
                wd1_ref, bd1_ref, wd2_ref, bd2_ref,
                wt0_ref, bt0_ref, wt1_ref, bt1_ref,
                xh_ref, p_ref):
    f32 = jnp.float32
    bf16 = jnp.bfloat16

    def mm(a, w_ref, b_ref, out_dtype):
        y = jnp.dot(a, w_ref[...], preferred_element_type=f32)
        return y.astype(out_dtype) + b_ref[...]

    def banded(a, w_ref, b_ref, chunks, out_dtype):
        parts = [
            jnp.dot(a[:, r0:r1], w_ref[r0:r1, c0:c0 + nc],
                    preferred_element_type=f32).astype(out_dtype)
            for (r0, r1, c0, nc) in chunks
        ]
        return jnp.concatenate(parts, axis=1) + b_ref[...]

    xb = x_ref[...].astype(bf16)
    h0 = _lrelu(banded(xb, wc0_ref, bc0_ref,
                       _CONV0_CHUNKS, f32)).astype(bf16)
    h1 = _lrelu(banded(h0, wc1_ref, bc1_ref,
                       _CONV1_CHUNKS, f32)).astype(bf16)
    h2 = _lrelu(mm(h1, wf1_ref, bf1_ref, f32)).astype(bf16)
    p = mm(h2, wf2_ref, bf2_ref, f32)
    p_ref[...] = p

    u = u_ref[...]
    g = -jnp.log(-jnp.log(u + _EPS) + _EPS)
    logits = (p + g) / _TEMP
    m = jnp.max(logits, axis=-1, keepdims=True)
    e = jnp.exp(logits - m)
    z = e / jnp.sum(e, axis=-1, keepdims=True)

    h3 = _lrelu(mm(z.astype(bf16), wd1_ref, bd1_ref, f32)).astype(bf16)
    h4 = _lrelu(mm(h3, wd2_ref, bd2_ref, f32)).astype(bf16)
    h5 = _lrelu(banded(h4, wt0_ref, bt0_ref,
                       _CONVT0_CHUNKS, f32)).astype(bf16)
    xh_ref[...] = banded(h5, wt1_ref, bt1_ref,
                         _CONVT1_CHUNKS, f32)


def _build_conv_matrices(enc_conv0_w, enc_conv1_w, enc_fc1_w,
                         dec_convt0_w, dec_convt1_w):
    bf16 = jnp.bfloat16

    def band(delta, taps):
        out = 0.
        for kk, tap in enumerate(taps):
            out = out + jnp.where(delta == kk, tap, 0.)
        return out.astype(bf16)

    i0 = jnp.arange(256)[:, None, None]
    l0 = jnp.arange(127)[None, :, None]
    wc0 = band(i0 - 2 * l0,
               [enc_conv0_w[None, None, :, 0, kk] for kk in range(3)])
    wc0 = wc0.reshape(256, 127 * 16)

    p1 = jnp.arange(127)[:, None, None, None]
    l1 = jnp.arange(63)[None, None, :, None]
    w1t = enc_conv1_w.transpose(1, 2, 0)
    wc1 = band(p1 - 2 * l1,
               [w1t[None, :, None, kk, :] for kk in range(3)])
    wc1 = wc1.reshape(127 * 16, 63 * 32)

    wf1 = (enc_fc1_w.reshape(32, 63, 512).transpose(1, 0, 2)
           .reshape(2016, 512).astype(bf16))

    lt = jnp.arange(63)[:, None, None, None]
    ot = jnp.arange(127)[None, None, :, None]
    wt0 = band(ot - 2 * lt,
               [dec_convt0_w[None, :, None, :, kk] for kk in range(3)])
    wt0 = wt0.reshape(32 * 63, 127 * 16)

    lt1 = jnp.arange(127)[:, None, None]
    ot1 = jnp.arange(256)[None, None, :]
    wt1 = band(ot1 - 2 * lt1,
               [dec_convt1_w[None, :, 0, kk, None] for kk in range(3)])
    wt1 = wt1.reshape(127 * 16, 256)

    return wc0, wc1, wf1, wt0, wt1


def kernel(x, noise_key, enc_conv0_w, enc_conv0_b, enc_conv1_w, enc_conv1_b,
           enc_fc1_w, enc_fc1_b, enc_fc2_w, enc_fc2_b, dec_fc1_w, dec_fc1_b,
           dec_fc2_w, dec_fc2_b, dec_convt0_w, dec_convt0_b, dec_convt1_w,
           dec_convt1_b):
    f32 = jnp.float32
    bf16 = jnp.bfloat16
    B = x.shape[0]

    wc0, wc1, wf1, wt0, wt1 = _build_conv_matrices(
        enc_conv0_w, enc_conv1_w, enc_fc1_w, dec_convt0_w, dec_convt1_w)

    bc0 = jnp.tile(enc_conv0_b, 127).reshape(1, 2032).astype(f32)
    bc1 = jnp.tile(enc_conv1_b, 63).reshape(1, 2016).astype(f32)
    bt0 = jnp.tile(dec_convt0_b, 127).reshape(1, 2032).astype(f32)
    bt1 = jnp.broadcast_to(dec_convt1_b.astype(f32), (256,)).reshape(1, 256)

    u = jax.random.uniform(noise_key, (B, 128), dtype=f32)

    weights = [
        wc0.astype(bf16), bc0,
        wc1.astype(bf16), bc1,
        wf1.astype(bf16), enc_fc1_b.reshape(1, 512).astype(f32),
        enc_fc2_w.astype(bf16), enc_fc2_b.reshape(1, 128).astype(f32),
        dec_fc1_w.astype(bf16), dec_fc1_b.reshape(1, 512).astype(f32),
        dec_fc2_w.reshape(512, 32, 63).transpose(0, 2, 1)
            .reshape(512, 2016).astype(bf16),
        dec_fc2_b.reshape(32, 63).transpose(1, 0).reshape(1, 2016).astype(f32),
        wt0.astype(bf16), bt0,
        wt1.astype(bf16), bt1,
    ]

    tb = min(512, B)
    assert B % tb == 0
    grid = (B // tb,)

    def row_spec(n):
        return pl.BlockSpec((tb, n), lambda i: (i, 0))

    def whole(a):
        return pl.BlockSpec(a.shape, lambda i: (0, 0))

    xh, p = pl.pallas_call(
        _fused_body,
        grid=grid,
        in_specs=[row_spec(256), row_spec(128)] + [whole(w) for w in weights],
        out_specs=[row_spec(256), row_spec(128)],
        out_shape=[jax.ShapeDtypeStruct((B, 256), f32),
                   jax.ShapeDtypeStruct((B, 128), f32)],
        compiler_params=pltpu.CompilerParams(
            dimension_semantics=("parallel",)),
    )(x, u, *weights)
    return xh, p


# speedup vs baseline: 1.0453x; 1.0453x over previous
import functools

import jax
import jax.numpy as jnp
from jax.experimental import pallas as pl
from jax.experimental.pallas import tpu as pltpu

_SLOPE = 0.01
_TEMP = 0.5
_EPS = 1e-7


def _lrelu(y):
    return jnp.maximum(y, y * jnp.asarray(_SLOPE, y.dtype))


def _conv0_chunks():
    out = []
    for l0 in range(0, 127, 16):
        nl = min(16, 127 - l0)
        out.append((2 * l0, min(2 * (l0 + nl - 1) + 3, 256),
                    16 * l0, 16 * nl))
    return out


def _conv1_chunks():
    out = []
    for l0 in range(0, 63, 4):
        nl = min(4, 63 - l0)
        out.append((32 * l0, min(32 * (l0 + nl) + 16, 2032),
                    32 * l0, 32 * nl))
    return out


def _convt0_chunks():
    out = []
    for o0 in range(0, 127, 8):
        no = min(8, 127 - o0)
        lmin = max(0, -((2 - o0) // 2))
        lmax = min(62, (o0 + no - 1) // 2)
        out.append((32 * lmin, 32 * (lmax + 1), 16 * o0, 16 * no))
    return out


_CONV0_CHUNKS = _conv0_chunks()
_CONV1_CHUNKS = _conv1_chunks()
_CONVT0_CHUNKS = _convt0_chunks()
_CONVT1_CHUNKS = [(0, 1024, 0, 128), (1008, 2032, 128, 128)]


def _fused_body(x_ref, u_ref,
                wc0_ref, bc0_ref, wc1_ref, bc1_ref,
                wf1_ref, bf1_ref, wf2_ref, bf2_ref,
                wd1_ref, bd1_ref, wd2_ref, bd2_ref,
                wt0_ref, bt0_ref, wt1_ref, bt1_ref,
                xh_ref, p_ref):
    f32 = jnp.float32
    bf16 = jnp.bfloat16

    def mm(a, w_ref, b_ref, out_dtype):
        y = jnp.dot(a, w_ref[...], preferred_element_type=f32)
        return y.astype(out_dtype) + b_ref[...]

    def banded(a, w_ref, b_ref, chunks, out_dtype):
        parts = [
            jnp.dot(a[:, r0:r1], w_ref[r0:r1, c0:c0 + nc],
                    preferred_element_type=f32).astype(out_dtype)
            for (r0, r1, c0, nc) in chunks
        ]
        return jnp.concatenate(parts, axis=1) + b_ref[...]

    xb = x_ref[...].astype(bf16)
    h0 = _lrelu(banded(xb, wc0_ref, bc0_ref,
                       _CONV0_CHUNKS, f32)).astype(bf16)
    h1 = _lrelu(banded(h0, wc1_ref, bc1_ref,
                       _CONV1_CHUNKS, f32)).astype(bf16)
    h2 = _lrelu(mm(h1, wf1_ref, bf1_ref, f32)).astype(bf16)
    p = mm(h2, wf2_ref, bf2_ref, f32)
    p_ref[...] = p

    u = u_ref[...]
    g = -jnp.log(-jnp.log(u + _EPS) + _EPS)
    logits = (p + g) / _TEMP
    m = jnp.max(logits, axis=-1, keepdims=True)
    e = jnp.exp(logits - m)
    z = e / jnp.sum(e, axis=-1, keepdims=True)

    h3 = _lrelu(mm(z.astype(bf16), wd1_ref, bd1_ref, f32)).astype(bf16)
    h4 = _lrelu(mm(h3, wd2_ref, bd2_ref, f32)).astype(bf16)
    h5 = _lrelu(banded(h4, wt0_ref, bt0_ref,
                       _CONVT0_CHUNKS, f32)).astype(bf16)
    xh_ref[...] = banded(h5, wt1_ref, bt1_ref,
                         _CONVT1_CHUNKS, f32)


def _build_conv_matrices(enc_conv0_w, enc_conv1_w, enc_fc1_w,
                         dec_convt0_w, dec_convt1_w):
    f32 = jnp.float32
    bf16 = jnp.bfloat16

    i0 = jnp.arange(256)[None, :, None]
    l0 = jnp.arange(127)[None, None, :]
    k = jnp.arange(3)[:, None, None]
    ind0 = (i0 == 2 * l0 + k).astype(f32)
    w0 = enc_conv0_w[:, 0, :].astype(f32)
    wc0 = jnp.einsum('kil,ok->ilo', ind0, w0).astype(bf16)
    wc0 = wc0.reshape(256, 127 * 16)

    p1 = jnp.arange(127)[None, :, None]
    l1 = jnp.arange(63)[None, None, :]
    ind1 = (p1 == 2 * l1 + k).astype(f32)
    wc1 = jnp.einsum('kpl,oik->pilo', ind1, enc_conv1_w.astype(f32))
    wc1 = wc1.astype(bf16).reshape(127 * 16, 63 * 32)

    wf1 = (enc_fc1_w.astype(bf16).reshape(32, 63, 512).transpose(1, 0, 2)
           .reshape(2016, 512))

    lt = jnp.arange(63)[None, :, None]
    ot = jnp.arange(127)[None, None, :]
    indt0 = (ot == 2 * lt + k).astype(f32)
    wt0 = jnp.einsum('klo,ick->lioc', indt0, dec_convt0_w.astype(f32))
    wt0 = wt0.astype(bf16).reshape(32 * 63, 127 * 16)

    lt1 = jnp.arange(127)[None, :, None]
    ot1 = jnp.arange(256)[None, None, :]
    indt1 = (ot1 == 2 * lt1 + k).astype(f32)
    wt1 = jnp.einsum('klo,ik->lio', indt1, dec_convt1_w[:, 0, :].astype(f32))
    wt1 = wt1.astype(bf16).reshape(127 * 16, 256)

    return wc0, wc1, wf1, wt0, wt1


def kernel(x, noise_key, enc_conv0_w, enc_conv0_b, enc_conv1_w, enc_conv1_b,
           enc_fc1_w, enc_fc1_b, enc_fc2_w, enc_fc2_b, dec_fc1_w, dec_fc1_b,
           dec_fc2_w, dec_fc2_b, dec_convt0_w, dec_convt0_b, dec_convt1_w,
           dec_convt1_b):
    f32 = jnp.float32
    bf16 = jnp.bfloat16
    B = x.shape[0]

    wc0, wc1, wf1, wt0, wt1 = _build_conv_matrices(
        enc_conv0_w, enc_conv1_w, enc_fc1_w, dec_convt0_w, dec_convt1_w)

    bc0 = jnp.tile(enc_conv0_b, 127).reshape(1, 2032).astype(f32)
    bc1 = jnp.tile(enc_conv1_b, 63).reshape(1, 2016).astype(f32)
    bt0 = jnp.tile(dec_convt0_b, 127).reshape(1, 2032).astype(f32)
    bt1 = jnp.broadcast_to(dec_convt1_b.astype(f32), (256,)).reshape(1, 256)

    u = jax.random.uniform(noise_key, (B, 128), dtype=f32)

    weights = [
        wc0.astype(bf16), bc0,
        wc1.astype(bf16), bc1,
        wf1.astype(bf16), enc_fc1_b.reshape(1, 512).astype(f32),
        enc_fc2_w.astype(bf16), enc_fc2_b.reshape(1, 128).astype(f32),
        dec_fc1_w.astype(bf16), dec_fc1_b.reshape(1, 512).astype(f32),
        dec_fc2_w.astype(bf16).reshape(512, 32, 63).transpose(0, 2, 1)
            .reshape(512, 2016),
        dec_fc2_b.reshape(32, 63).transpose(1, 0).reshape(1, 2016).astype(f32),
        wt0.astype(bf16), bt0,
        wt1.astype(bf16), bt1,
    ]

    tb = min(512, B)
    assert B % tb == 0
    grid = (B // tb,)

    def row_spec(n):
        return pl.BlockSpec((tb, n), lambda i: (i, 0))

    def whole(a):
        return pl.BlockSpec(a.shape, lambda i: (0, 0))

    xh, p = pl.pallas_call(
        _fused_body,
        grid=grid,
        in_specs=[row_spec(256), row_spec(128)] + [whole(w) for w in weights],
        out_specs=[row_spec(256), row_spec(128)],
        out_shape=[jax.ShapeDtypeStruct((B, 256), f32),
                   jax.ShapeDtypeStruct((B, 128), f32)],
        compiler_params=pltpu.CompilerParams(
            dimension_semantics=("parallel",)),
    )(x, u, *weights)
    return xh, p


# speedup vs baseline: 1.5002x; 1.4351x over previous
import functools

import jax
import jax.numpy as jnp
from jax.experimental import pallas as pl
from jax.experimental.pallas import tpu as pltpu

_SLOPE = 0.01
_TEMP = 0.5
_EPS = 1e-7


def _lrelu(y):
    return jnp.maximum(y, y * jnp.asarray(_SLOPE, y.dtype))


def _conv0_chunks():
    out = []
    for l0 in range(0, 127, 16):
        nl = min(16, 127 - l0)
        out.append((2 * l0, min(2 * (l0 + nl - 1) + 3, 256),
                    16 * l0, 16 * nl))
    return out


def _conv1_chunks():
    out = []
    for l0 in range(0, 63, 4):
        nl = min(4, 63 - l0)
        out.append((32 * l0, min(32 * (l0 + nl) + 16, 2032),
                    32 * l0, 32 * nl))
    return out


def _convt0_chunks():
    out = []
    for o0 in range(0, 127, 8):
        no = min(8, 127 - o0)
        lmin = max(0, -((2 - o0) // 2))
        lmax = min(62, (o0 + no - 1) // 2)
        out.append((32 * lmin, 32 * (lmax + 1), 16 * o0, 16 * no))
    return out


_CONV0_CHUNKS = _conv0_chunks()
_CONV1_CHUNKS = _conv1_chunks()
_CONVT0_CHUNKS = _convt0_chunks()
_CONVT1_CHUNKS = [(0, 1024, 0, 128), (1008, 2032, 128, 128)]


def _fused_body(x_ref, u_ref,
                wc0_ref, bc0_ref, w1tap_ref, bc1_ref,
                wf1_ref, bf1_ref, wf2_ref, bf2_ref,
                wd1_ref, bd1_ref, wd2_ref, bd2_ref,
                wt0tap_ref, bt0_ref, wt1_ref, bt1_ref,
                xh_ref, p_ref,
                wc1_s, wt0_s):
    f32 = jnp.float32
    bf16 = jnp.bfloat16

    @pl.when(pl.program_id(0) == 0)
    def _build():
        wc1_s[...] = jnp.zeros((2032, 2016), bf16)
        wt0_s[...] = jnp.zeros((2016, 2032), bf16)
        tap1 = w1tap_ref[...]
        tap0 = wt0tap_ref[...]
        for l in range(63):
            wc1_s[32 * l:32 * l + 48, 32 * l:32 * l + 32] = tap1
            wt0_s[32 * l:32 * l + 32, 32 * l:32 * l + 48] = tap0

    def mm(a, w_ref, b_ref, out_dtype):
        y = jnp.dot(a, w_ref[...], preferred_element_type=f32)
        return y.astype(out_dtype) + b_ref[...]

    def banded(a, w_ref, b_ref, chunks, out_dtype):
        parts = [
            jnp.dot(a[:, r0:r1], w_ref[r0:r1, c0:c0 + nc],
                    preferred_element_type=f32).astype(out_dtype)
            for (r0, r1, c0, nc) in chunks
        ]
        return jnp.concatenate(parts, axis=1) + b_ref[...]

    xb = x_ref[...].astype(bf16)
    h0 = _lrelu(banded(xb, wc0_ref, bc0_ref,
                       _CONV0_CHUNKS, f32)).astype(bf16)
    h1 = _lrelu(banded(h0, wc1_s, bc1_ref,
                       _CONV1_CHUNKS, f32)).astype(bf16)
    h2 = _lrelu(mm(h1, wf1_ref, bf1_ref, f32)).astype(bf16)
    p = mm(h2, wf2_ref, bf2_ref, f32)
    p_ref[...] = p

    u = u_ref[...]
    g = -jnp.log(-jnp.log(u + _EPS) + _EPS)
    logits = (p + g) / _TEMP
    m = jnp.max(logits, axis=-1, keepdims=True)
    e = jnp.exp(logits - m)
    z = e / jnp.sum(e, axis=-1, keepdims=True)

    h3 = _lrelu(mm(z.astype(bf16), wd1_ref, bd1_ref, f32)).astype(bf16)
    h4 = _lrelu(mm(h3, wd2_ref, bd2_ref, f32)).astype(bf16)
    h5 = _lrelu(banded(h4, wt0_s, bt0_ref,
                       _CONVT0_CHUNKS, f32)).astype(bf16)
    xh_ref[...] = banded(h5, wt1_ref, bt1_ref,
                         _CONVT1_CHUNKS, f32)


def _build_conv_matrices(enc_conv0_w, enc_conv1_w, enc_fc1_w,
                         dec_convt0_w, dec_convt1_w):
    f32 = jnp.float32
    bf16 = jnp.bfloat16

    i0 = jnp.arange(256)[None, :, None]
    l0 = jnp.arange(127)[None, None, :]
    k = jnp.arange(3)[:, None, None]
    ind0 = (i0 == 2 * l0 + k).astype(f32)
    w0 = enc_conv0_w[:, 0, :].astype(f32)
    wc0 = jnp.einsum('kil,ok->ilo', ind0, w0).astype(bf16)
    wc0 = wc0.reshape(256, 127 * 16)

    w1tap = enc_conv1_w.astype(bf16).transpose(2, 1, 0).reshape(48, 32)

    wf1 = (enc_fc1_w.astype(bf16).reshape(32, 63, 512).transpose(1, 0, 2)
           .reshape(2016, 512))

    wt0tap = dec_convt0_w.astype(bf16).transpose(0, 2, 1).reshape(32, 48)

    lt1 = jnp.arange(127)[None, :, None]
    ot1 = jnp.arange(256)[None, None, :]
    indt1 = (ot1 == 2 * lt1 + k).astype(f32)
    wt1 = jnp.einsum('klo,ik->lio', indt1, dec_convt1_w[:, 0, :].astype(f32))
    wt1 = wt1.astype(bf16).reshape(127 * 16, 256)

    return wc0, w1tap, wf1, wt0tap, wt1


def kernel(x, noise_key, enc_conv0_w, enc_conv0_b, enc_conv1_w, enc_conv1_b,
           enc_fc1_w, enc_fc1_b, enc_fc2_w, enc_fc2_b, dec_fc1_w, dec_fc1_b,
           dec_fc2_w, dec_fc2_b, dec_convt0_w, dec_convt0_b, dec_convt1_w,
           dec_convt1_b):
    f32 = jnp.float32
    bf16 = jnp.bfloat16
    B = x.shape[0]

    wc0, w1tap, wf1, wt0tap, wt1 = _build_conv_matrices(
        enc_conv0_w, enc_conv1_w, enc_fc1_w, dec_convt0_w, dec_convt1_w)

    bc0 = jnp.tile(enc_conv0_b, 127).reshape(1, 2032).astype(f32)
    bc1 = jnp.tile(enc_conv1_b, 63).reshape(1, 2016).astype(f32)
    bt0 = jnp.tile(dec_convt0_b, 127).reshape(1, 2032).astype(f32)
    bt1 = jnp.broadcast_to(dec_convt1_b.astype(f32), (256,)).reshape(1, 256)

    u = jax.random.uniform(noise_key, (B, 128), dtype=f32)

    weights = [
        wc0, bc0,
        w1tap, bc1,
        wf1, enc_fc1_b.reshape(1, 512).astype(f32),
        enc_fc2_w.astype(bf16), enc_fc2_b.reshape(1, 128).astype(f32),
        dec_fc1_w.astype(bf16), dec_fc1_b.reshape(1, 512).astype(f32),
        dec_fc2_w.astype(bf16).reshape(512, 32, 63).transpose(0, 2, 1)
            .reshape(512, 2016),
        dec_fc2_b.reshape(32, 63).transpose(1, 0).reshape(1, 2016).astype(f32),
        wt0tap, bt0,
        wt1, bt1,
    ]

    tb = min(512, B)
    assert B % tb == 0
    grid = (B // tb,)

    def row_spec(n):
        return pl.BlockSpec((tb, n), lambda i: (i, 0))

    def whole(a):
        return pl.BlockSpec(a.shape, lambda i: (0, 0))

    xh, p = pl.pallas_call(
        _fused_body,
        grid=grid,
        in_specs=[row_spec(256), row_spec(128)] + [whole(w) for w in weights],
        out_specs=[row_spec(256), row_spec(128)],
        out_shape=[jax.ShapeDtypeStruct((B, 256), f32),
                   jax.ShapeDtypeStruct((B, 128), f32)],
        scratch_shapes=[pltpu.VMEM((2032, 2016), bf16),
                        pltpu.VMEM((2016, 2032), bf16)],
        compiler_params=pltpu.CompilerParams(
            dimension_semantics=("arbitrary",)),
    )(x, u, *weights)
    return xh, p


# speedup vs baseline: 1.5524x; 1.0348x over previous
import functools

import jax
import jax.numpy as jnp
from jax.experimental import pallas as pl
from jax.experimental.pallas import tpu as pltpu

_SLOPE = 0.01
_TEMP = 0.5
_EPS = 1e-7


def _lrelu(y):
    return jnp.maximum(y, y * jnp.asarray(_SLOPE, y.dtype))


def _conv0_chunks():
    out = []
    for l0 in range(0, 127, 16):
        nl = min(16, 127 - l0)
        out.append((2 * l0, min(2 * (l0 + nl - 1) + 3, 256),
                    16 * l0, 16 * nl))
    return out


def _conv1_chunks():
    out = []
    for l0 in range(0, 63, 4):
        nl = min(4, 63 - l0)
        out.append((32 * l0, min(32 * (l0 + nl) + 16, 2032),
                    32 * l0, 32 * nl))
    return out


def _convt0_chunks():
    out = []
    for o0 in range(0, 127, 8):
        no = min(8, 127 - o0)
        lmin = max(0, -((2 - o0) // 2))
        lmax = min(62, (o0 + no - 1) // 2)
        out.append((32 * lmin, 32 * (lmax + 1), 16 * o0, 16 * no))
    return out


_CONV0_CHUNKS = _conv0_chunks()
_CONV1_CHUNKS = _conv1_chunks()
_CONVT0_CHUNKS = _convt0_chunks()
_CONVT1_CHUNKS = [(0, 1024, 0, 128), (1008, 2032, 128, 128)]


def _fused_body(x_ref, u_ref,
                wc0_ref, bc0_ref, w1tap_ref, bc1_ref,
                wf1_ref, bf1_ref, wf2_ref, bf2_ref,
                wd1_ref, bd1_ref, wd2_ref, bd2_ref,
                wt0tap_ref, bt0_ref, wt1_ref, bt1_ref,
                xh_ref, p_ref,
                wc1_s, wt0_s):
    f32 = jnp.float32
    bf16 = jnp.bfloat16

    @pl.when(pl.program_id(0) == 0)
    def _build():
        wc1_s[...] = jnp.zeros((2032, 2016), bf16)
        wt0_s[...] = jnp.zeros((2016, 2032), bf16)
        tap1 = w1tap_ref[...]
        tap0 = wt0tap_ref[...]
        for l in range(63):
            wc1_s[32 * l:32 * l + 48, 32 * l:32 * l + 32] = tap1
            wt0_s[32 * l:32 * l + 32, 32 * l:32 * l + 48] = tap0

    def mm(a, w_ref, b_ref, out_dtype):
        y = jnp.dot(a, w_ref[...], preferred_element_type=f32)
        return y.astype(out_dtype) + b_ref[...]

    def banded(a, w_ref, b_ref, chunks, out_dtype):
        parts = [
            jnp.dot(a[:, r0:r1], w_ref[r0:r1, c0:c0 + nc],
                    preferred_element_type=f32).astype(out_dtype)
            for (r0, r1, c0, nc) in chunks
        ]
        return jnp.concatenate(parts, axis=1) + b_ref[...]

    xb = x_ref[...].astype(bf16)
    h0 = _lrelu(banded(xb, wc0_ref, bc0_ref,
                       _CONV0_CHUNKS, f32)).astype(bf16)
    h1 = _lrelu(banded(h0, wc1_s, bc1_ref,
                       _CONV1_CHUNKS, f32)).astype(bf16)
    h2 = _lrelu(mm(h1, wf1_ref, bf1_ref, f32)).astype(bf16)
    p = mm(h2, wf2_ref, bf2_ref, f32)
    p_ref[...] = p

    u = u_ref[...]
    g = -jnp.log(-jnp.log(u + _EPS) + _EPS)
    logits = (p + g) / _TEMP
    m = jnp.max(logits, axis=-1, keepdims=True)
    e = jnp.exp(logits - m)
    z = e / jnp.sum(e, axis=-1, keepdims=True)

    h3 = _lrelu(mm(z.astype(bf16), wd1_ref, bd1_ref, f32)).astype(bf16)
    h4 = _lrelu(mm(h3, wd2_ref, bd2_ref, f32)).astype(bf16)
    h5 = _lrelu(banded(h4, wt0_s, bt0_ref,
                       _CONVT0_CHUNKS, f32)).astype(bf16)
    xh_ref[...] = banded(h5, wt1_ref, bt1_ref,
                         _CONVT1_CHUNKS, f32)


def _build_conv_matrices(enc_conv0_w, enc_conv1_w, enc_fc1_w,
                         dec_convt0_w, dec_convt1_w):
    f32 = jnp.float32
    bf16 = jnp.bfloat16

    i0 = jnp.arange(256)[None, :, None]
    l0 = jnp.arange(127)[None, None, :]
    k = jnp.arange(3)[:, None, None]
    ind0 = (i0 == 2 * l0 + k).astype(f32)
    w0 = enc_conv0_w[:, 0, :].astype(f32)
    wc0 = jnp.einsum('kil,ok->ilo', ind0, w0).astype(bf16)
    wc0 = wc0.reshape(256, 127 * 16)

    w1tap = enc_conv1_w.astype(bf16).transpose(2, 1, 0).reshape(48, 32)

    wf1 = (enc_fc1_w.astype(bf16).reshape(32, 63, 512).transpose(1, 0, 2)
           .reshape(2016, 512))

    wt0tap = dec_convt0_w.astype(bf16).transpose(0, 2, 1).reshape(32, 48)

    lt1 = jnp.arange(127)[None, :, None]
    ot1 = jnp.arange(256)[None, None, :]
    indt1 = (ot1 == 2 * lt1 + k).astype(f32)
    wt1 = jnp.einsum('klo,ik->lio', indt1, dec_convt1_w[:, 0, :].astype(f32))
    wt1 = wt1.astype(bf16).reshape(127 * 16, 256)

    return wc0, w1tap, wf1, wt0tap, wt1


def kernel(x, noise_key, enc_conv0_w, enc_conv0_b, enc_conv1_w, enc_conv1_b,
           enc_fc1_w, enc_fc1_b, enc_fc2_w, enc_fc2_b, dec_fc1_w, dec_fc1_b,
           dec_fc2_w, dec_fc2_b, dec_convt0_w, dec_convt0_b, dec_convt1_w,
           dec_convt1_b):
    f32 = jnp.float32
    bf16 = jnp.bfloat16
    B = x.shape[0]

    wc0, w1tap, wf1, wt0tap, wt1 = _build_conv_matrices(
        enc_conv0_w, enc_conv1_w, enc_fc1_w, dec_convt0_w, dec_convt1_w)

    bc0 = jnp.tile(enc_conv0_b, 127).reshape(1, 2032).astype(f32)
    bc1 = jnp.tile(enc_conv1_b, 63).reshape(1, 2016).astype(f32)
    bt0 = jnp.tile(dec_convt0_b, 127).reshape(1, 2032).astype(f32)
    bt1 = jnp.broadcast_to(dec_convt1_b.astype(f32), (256,)).reshape(1, 256)

    u = jax.random.uniform(noise_key, (B, 128), dtype=f32)

    weights = [
        wc0, bc0,
        w1tap, bc1,
        wf1, enc_fc1_b.reshape(1, 512).astype(f32),
        enc_fc2_w.astype(bf16), enc_fc2_b.reshape(1, 128).astype(f32),
        dec_fc1_w.astype(bf16), dec_fc1_b.reshape(1, 512).astype(f32),
        dec_fc2_w.astype(bf16).reshape(512, 32, 63).transpose(0, 2, 1)
            .reshape(512, 2016),
        dec_fc2_b.reshape(32, 63).transpose(1, 0).reshape(1, 2016).astype(f32),
        wt0tap, bt0,
        wt1, bt1,
    ]

    tb = min(1024, B)
    assert B % tb == 0
    grid = (B // tb,)

    def row_spec(n):
        return pl.BlockSpec((tb, n), lambda i: (i, 0))

    def whole(a):
        return pl.BlockSpec(a.shape, lambda i: (0, 0))

    xh, p = pl.pallas_call(
        _fused_body,
        grid=grid,
        in_specs=[row_spec(256), row_spec(128)] + [whole(w) for w in weights],
        out_specs=[row_spec(256), row_spec(128)],
        out_shape=[jax.ShapeDtypeStruct((B, 256), f32),
                   jax.ShapeDtypeStruct((B, 128), f32)],
        scratch_shapes=[pltpu.VMEM((2032, 2016), bf16),
                        pltpu.VMEM((2016, 2032), bf16)],
        compiler_params=pltpu.CompilerParams(
            dimension_semantics=("arbitrary",)),
    )(x, u, *weights)
    return xh, p


# speedup vs baseline: 1.6293x; 1.0495x over previous
import functools

import jax
import jax.numpy as jnp
from jax import lax
from jax.experimental import pallas as pl
from jax.experimental.pallas import tpu as pltpu

_SLOPE = 0.01
_TEMP = 0.5
_EPS = 1e-7


def _lrelu(y):
    return jnp.maximum(y, y * jnp.asarray(_SLOPE, y.dtype))


def _conv0_chunks():
    out = []
    for l0 in range(0, 127, 16):
        nl = min(16, 127 - l0)
        out.append((2 * l0, min(2 * (l0 + nl - 1) + 3, 256),
                    16 * l0, 16 * nl))
    return out


def _conv1_chunks():
    out = []
    for l0 in range(0, 63, 4):
        nl = min(4, 63 - l0)
        out.append((32 * l0, min(32 * (l0 + nl) + 16, 2032),
                    32 * l0, 32 * nl))
    return out


def _convt0_chunks():
    out = []
    for o0 in range(0, 127, 8):
        no = min(8, 127 - o0)
        lmin = max(0, -((2 - o0) // 2))
        lmax = min(62, (o0 + no - 1) // 2)
        out.append((32 * lmin, 32 * (lmax + 1), 16 * o0, 16 * no))
    return out


_CONV0_CHUNKS = _conv0_chunks()
_CONV1_CHUNKS = _conv1_chunks()
_CONVT0_CHUNKS = _convt0_chunks()
_CONVT1_CHUNKS = [(0, 1024, 0, 128), (1008, 2032, 128, 128)]


def _threefry_uniform(key_ref, tb, n_total):
    u32 = jnp.uint32
    k0 = key_ref[0, 0]
    k1 = key_ref[0, 1]
    ks2 = k0 ^ k1 ^ jnp.asarray(0x1BD11BDA, u32)

    base = (pl.program_id(0) * (tb * 128)).astype(u32)
    jj = (base
          + lax.broadcasted_iota(u32, (tb, 128), 0) * jnp.asarray(128, u32)
          + lax.broadcasted_iota(u32, (tb, 128), 1))
    x0 = jnp.zeros((tb, 128), u32) + k0
    x1 = jj + k1

    def rotl(x, d):
        return lax.shift_left(x, jnp.asarray(d, u32)) | lax.shift_right_logical(
            x, jnp.asarray(32 - d, u32))

    def four(x0, x1, rots):
        for r in rots:
            x0 = x0 + x1
            x1 = rotl(x1, r)
            x1 = x1 ^ x0
        return x0, x1

    ra, rb = (13, 15, 26, 6), (17, 29, 16, 24)
    one = jnp.asarray(1, u32)
    x0, x1 = four(x0, x1, ra)
    x0, x1 = x0 + k1, x1 + ks2 + one
    x0, x1 = four(x0, x1, rb)
    x0, x1 = x0 + ks2, x1 + k0 + 2 * one
    x0, x1 = four(x0, x1, ra)
    x0, x1 = x0 + k0, x1 + k1 + 3 * one
    x0, x1 = four(x0, x1, rb)
    x0, x1 = x0 + k1, x1 + ks2 + 4 * one
    x0, x1 = four(x0, x1, ra)
    x0, x1 = x0 + ks2, x1 + k0 + 5 * one

    bits = x0 ^ x1
    fbits = lax.shift_right_logical(bits, jnp.asarray(9, u32)) | jnp.asarray(
        0x3F800000, u32)
    return lax.bitcast_convert_type(fbits, jnp.float32) - 1.0


def _fused_body(x_ref, key_ref,
                wc0_ref, bc0_ref, w1tap_ref, bc1_ref,
                wf1_ref, bf1_ref, wf2_ref, bf2_ref,
                wd1_ref, bd1_ref, wd2_ref, bd2_ref,
                wt0tap_ref, bt0_ref, wt1_ref, bt1_ref,
                xh_ref, p_ref,
                wc1_s, wt0_s, *, n_total):
    f32 = jnp.float32
    bf16 = jnp.bfloat16

    @pl.when(pl.program_id(0) == 0)
    def _build():
        wc1_s[...] = jnp.zeros((2032, 2016), bf16)
        wt0_s[...] = jnp.zeros((2016, 2032), bf16)
        tap1 = w1tap_ref[...]
        tap0 = wt0tap_ref[...]
        for l in range(63):
            wc1_s[32 * l:32 * l + 48, 32 * l:32 * l + 32] = tap1
            wt0_s[32 * l:32 * l + 32, 32 * l:32 * l + 48] = tap0

    def mm(a, w_ref, b_ref, out_dtype):
        y = jnp.dot(a, w_ref[...], preferred_element_type=f32)
        return y.astype(out_dtype) + b_ref[...]

    def banded(a, w_ref, b_ref, chunks, out_dtype):
        parts = [
            jnp.dot(a[:, r0:r1], w_ref[r0:r1, c0:c0 + nc],
                    preferred_element_type=f32).astype(out_dtype)
            for (r0, r1, c0, nc) in chunks
        ]
        return jnp.concatenate(parts, axis=1) + b_ref[...]

    xb = x_ref[...].astype(bf16)
    h0 = _lrelu(banded(xb, wc0_ref, bc0_ref,
                       _CONV0_CHUNKS, f32)).astype(bf16)
    h1 = _lrelu(banded(h0, wc1_s, bc1_ref,
                       _CONV1_CHUNKS, f32)).astype(bf16)
    h2 = _lrelu(mm(h1, wf1_ref, bf1_ref, f32)).astype(bf16)
    p = mm(h2, wf2_ref, bf2_ref, f32)
    p_ref[...] = p

    u = _threefry_uniform(key_ref, x_ref.shape[0], n_total)
    g = -jnp.log(-jnp.log(u + _EPS) + _EPS)
    logits = (p + g) / _TEMP
    m = jnp.max(logits, axis=-1, keepdims=True)
    e = jnp.exp(logits - m)
    z = e / jnp.sum(e, axis=-1, keepdims=True)

    h3 = _lrelu(mm(z.astype(bf16), wd1_ref, bd1_ref, f32)).astype(bf16)
    h4 = _lrelu(mm(h3, wd2_ref, bd2_ref, f32)).astype(bf16)
    h5 = _lrelu(banded(h4, wt0_s, bt0_ref,
                       _CONVT0_CHUNKS, f32)).astype(bf16)
    xh_ref[...] = banded(h5, wt1_ref, bt1_ref,
                         _CONVT1_CHUNKS, f32)


def _build_conv_matrices(enc_conv0_w, enc_conv1_w, enc_fc1_w,
                         dec_convt0_w, dec_convt1_w):
    f32 = jnp.float32
    bf16 = jnp.bfloat16

    i0 = jnp.arange(256)[None, :, None]
    l0 = jnp.arange(127)[None, None, :]
    k = jnp.arange(3)[:, None, None]
    ind0 = (i0 == 2 * l0 + k).astype(f32)
    w0 = enc_conv0_w[:, 0, :].astype(f32)
    wc0 = jnp.einsum('kil,ok->ilo', ind0, w0).astype(bf16)
    wc0 = wc0.reshape(256, 127 * 16)

    w1tap = enc_conv1_w.astype(bf16).transpose(2, 1, 0).reshape(48, 32)

    wf1 = (enc_fc1_w.astype(bf16).reshape(32, 63, 512).transpose(1, 0, 2)
           .reshape(2016, 512))

    wt0tap = dec_convt0_w.astype(bf16).transpose(0, 2, 1).reshape(32, 48)

    lt1 = jnp.arange(127)[None, :, None]
    ot1 = jnp.arange(256)[None, None, :]
    indt1 = (ot1 == 2 * lt1 + k).astype(f32)
    wt1 = jnp.einsum('klo,ik->lio', indt1, dec_convt1_w[:, 0, :].astype(f32))
    wt1 = wt1.astype(bf16).reshape(127 * 16, 256)

    return wc0, w1tap, wf1, wt0tap, wt1


def kernel(x, noise_key, enc_conv0_w, enc_conv0_b, enc_conv1_w, enc_conv1_b,
           enc_fc1_w, enc_fc1_b, enc_fc2_w, enc_fc2_b, dec_fc1_w, dec_fc1_b,
           dec_fc2_w, dec_fc2_b, dec_convt0_w, dec_convt0_b, dec_convt1_w,
           dec_convt1_b):
    f32 = jnp.float32
    bf16 = jnp.bfloat16
    B = x.shape[0]

    wc0, w1tap, wf1, wt0tap, wt1 = _build_conv_matrices(
        enc_conv0_w, enc_conv1_w, enc_fc1_w, dec_convt0_w, dec_convt1_w)

    bc0 = jnp.tile(enc_conv0_b, 127).reshape(1, 2032).astype(f32)
    bc1 = jnp.tile(enc_conv1_b, 63).reshape(1, 2016).astype(f32)
    bt0 = jnp.tile(dec_convt0_b, 127).reshape(1, 2032).astype(f32)
    bt1 = jnp.broadcast_to(dec_convt1_b.astype(f32), (256,)).reshape(1, 256)


    weights = [
        wc0, bc0,
        w1tap, bc1,
        wf1, enc_fc1_b.reshape(1, 512).astype(f32),
        enc_fc2_w.astype(bf16), enc_fc2_b.reshape(1, 128).astype(f32),
        dec_fc1_w.astype(bf16), dec_fc1_b.reshape(1, 512).astype(f32),
        dec_fc2_w.astype(bf16).reshape(512, 32, 63).transpose(0, 2, 1)
            .reshape(512, 2016),
        dec_fc2_b.reshape(32, 63).transpose(1, 0).reshape(1, 2016).astype(f32),
        wt0tap, bt0,
        wt1, bt1,
    ]

    tb = min(1024, B)
    assert B % tb == 0
    grid = (B // tb,)

    def row_spec(n):
        return pl.BlockSpec((tb, n), lambda i: (i, 0))

    def whole(a):
        return pl.BlockSpec(a.shape, lambda i: (0, 0))

    xh, p = pl.pallas_call(
        functools.partial(_fused_body, n_total=B * 128),
        grid=grid,
        in_specs=[row_spec(256), pl.BlockSpec((1, 2), lambda i: (0, 0))]
        + [whole(w) for w in weights],
        out_specs=[row_spec(256), row_spec(128)],
        out_shape=[jax.ShapeDtypeStruct((B, 256), f32),
                   jax.ShapeDtypeStruct((B, 128), f32)],
        scratch_shapes=[pltpu.VMEM((2032, 2016), bf16),
                        pltpu.VMEM((2016, 2032), bf16)],
        compiler_params=pltpu.CompilerParams(
            dimension_semantics=("arbitrary",)),
    )(x, noise_key.reshape(1, 2), *weights)
    return xh, p
